# baseline (speedup 1.0000x reference)
import functools

import jax
import jax.numpy as jnp
from jax.experimental import pallas as pl
from jax.experimental.pallas import tpu as pltpu

_KS = 1024
_KP = 512
_CH = 256


def _search_init(cols):
    return (jnp.zeros((1, cols), jnp.int32),
            jnp.full((1, cols), 0x7F7FFFFF, jnp.int32))


def _search_step(bits, k, lo, hi):
    mid = lo + (hi - lo + 1) // 2
    cnt = jnp.sum((bits >= mid).astype(jnp.int32), axis=0, keepdims=True)
    ok = cnt >= k
    return jnp.where(ok, mid, lo), jnp.where(ok, hi, mid - 1)


def _fused_body(k_s, k_p, nblk,
                state_ref, action_ref, W1s_ref, W1a_ref, b1_ref, W2_ref,
                b2_ref, Wcat_ref, bcat_ref, Wa2_ref, ba2_ref,
                DsT_ref, DpT_ref,
                next_ref, alpha_ref,
                h2_ref, lcat_ref, lp_ref, btsT_ref, btpT_ref, sprev_ref):
    i = pl.program_id(0)
    p = jax.lax.rem(i, 2)
    pp = 1 - p
    R = state_ref.shape[0]

    def search2(carry):
        lo_s, hi_s, lo_p, hi_p = carry
        for _ in range(2):
            lo_s, hi_s = _search_step(btsT_ref[pp], k_s, lo_s, hi_s)
            lo_p, hi_p = _search_step(btpT_ref[pp], k_p, lo_p, hi_p)
        return lo_s, hi_s, lo_p, hi_p

    s = state_ref[...]
    h1 = s @ W1s_ref[...] + action_ref[...] @ W1a_ref[...] + b1_ref[...]
    h1 = jnp.maximum(h1, 0.0)

    carry = _search_init(R) + _search_init(R)

    def bodyA(j, carry):
        c = j * _CH
        h2_ref[:, pl.ds(c, _CH)] = jnp.maximum(
            h1 @ W2_ref[:, pl.ds(c, _CH)] + b2_ref[:, pl.ds(c, _CH)], 0.0)
        return search2(carry)

    carry = jax.lax.fori_loop(0, 8, bodyA, carry)
    h2 = h2_ref[...]

    def bodyB(j, carry):
        c = j * _CH
        lcat_ref[p, :, pl.ds(c, _CH)] = (
            h2 @ Wcat_ref[:, pl.ds(c, _CH)] + bcat_ref[:, pl.ds(c, _CH)])
        return search2(carry)

    carry = jax.lax.fori_loop(0, 6, bodyB, carry)
    ah = jnp.maximum(lcat_ref[p, :, _KS:], 0.0)

    def bodyC(j, carry):
        c = j * _CH
        lp_ref[p, :, pl.ds(c, _CH)] = (
            ah @ Wa2_ref[:, pl.ds(c, _CH)] + ba2_ref[:, pl.ds(c, _CH)])
        return search2(carry)

    lo_s, _, lo_p, _ = jax.lax.fori_loop(0, 2, bodyC, carry)

    ls_prev = lcat_ref[pp, :, :_KS]
    lp_prev = lp_ref[pp]
    bits_s = jax.lax.bitcast_convert_type(jnp.abs(ls_prev), jnp.int32)
    bits_p = jax.lax.bitcast_convert_type(jnp.abs(lp_prev), jnp.int32)
    alpha_s = jnp.where(bits_s >= lo_s.T, ls_prev, 0.0)
    alpha_p = jnp.where(bits_p >= lo_p.T, lp_prev, 0.0)

    delta = alpha_s @ DsT_ref[...] + alpha_p @ DpT_ref[...]
    next_ref[...] = sprev_ref[pp] + delta
    alpha_ref[:, :_KS] = alpha_s
    alpha_ref[:, _KS:] = alpha_p

    sprev_ref[p] = s
    btsT_ref[p] = jax.lax.bitcast_convert_type(
        jnp.abs(lcat_ref[p, :, :_KS]), jnp.int32).T
    btpT_ref[p] = jax.lax.bitcast_convert_type(
        jnp.abs(lp_ref[p]), jnp.int32).T


def kernel(state, action, W1, b1, W2, b2, Ws, bs, Wa1, ba1, Wa2, ba2, Ds, Dp):
    B, S = state.shape
    A = action.shape[1]
    H = W1.shape[1]
    KS = Ws.shape[1]
    KP = Wa2.shape[1]
    AD = Wa1.shape[1]
    R = 256
    assert B % R == 0
    nblk = B // R

    W1s = W1[:S]
    W1a = W1[S:]
    Wcat = jnp.concatenate([Ws, Wa1], axis=1)
    bcat = jnp.concatenate([bs, ba1]).reshape(1, KS + AD)
    DsT = Ds.T
    DpT = Dp.T

    full = lambda shape: pl.BlockSpec(shape, lambda i: (0, 0))
    last = nblk - 1
    cur = lambda i: (jnp.minimum(i, last), 0)
    prev = lambda i: (jnp.maximum(i - 1, 0), 0)
    grid = (nblk + 1,)
    in_specs = [
            pl.BlockSpec((R, S), cur),
            pl.BlockSpec((R, A), cur),
            full((S, H)),
            full((A, H)),
            full((1, H)),
            full((H, H)),
            full((1, H)),
            full((H, KS + AD)),
            full((1, KS + AD)),
            full((AD, KP)),
            full((1, KP)),
            full((KS, S)),
            full((KP, S)),
    ]
    out_specs = [
        pl.BlockSpec((R, S), prev),
        pl.BlockSpec((R, KS + KP), prev),
    ]

    next_state, alpha = pl.pallas_call(
        functools.partial(_fused_body, 64, 64, nblk),
        grid=grid,
        in_specs=in_specs,
        out_specs=out_specs,
        out_shape=[
            jax.ShapeDtypeStruct((B, S), jnp.float32),
            jax.ShapeDtypeStruct((B, KS + KP), jnp.float32),
        ],
        scratch_shapes=[
            pltpu.VMEM((R, H), jnp.float32),
            pltpu.VMEM((2, R, KS + AD), jnp.float32),
            pltpu.VMEM((2, R, KP), jnp.float32),
            pltpu.VMEM((2, KS, R), jnp.int32),
            pltpu.VMEM((2, KP, R), jnp.int32),
            pltpu.VMEM((2, R, S), jnp.float32),
        ],
        compiler_params=pltpu.CompilerParams(
            dimension_semantics=("arbitrary",),
        ),
    )(state, action, W1s, W1a, b1.reshape(1, H), W2, b2.reshape(1, H),
      Wcat, bcat, Wa2, ba2.reshape(1, KP), DsT, DpT)
    return (next_state, alpha)


# speedup vs baseline: 1.0656x; 1.0656x over previous
import functools

import jax
import jax.numpy as jnp
from jax.experimental import pallas as pl
from jax.experimental.pallas import tpu as pltpu

_KS = 1024
_KP = 512
_CH = 512


def _search_init(cols):
    return (jnp.zeros((1, cols), jnp.int32),
            jnp.full((1, cols), 0x7F7FFFFF, jnp.int32))


def _search_step(bits, k, lo, hi):
    mid = lo + (hi - lo + 1) // 2
    cnt = jnp.sum((bits >= mid).astype(jnp.int32), axis=0, keepdims=True)
    ok = cnt >= k
    return jnp.where(ok, mid, lo), jnp.where(ok, hi, mid - 1)


def _fused_body(k_s, k_p, nblk,
                state_ref, action_ref, W1s_ref, W1a_ref, b1_ref, W2_ref,
                b2_ref, Wcat_ref, bcat_ref, Wa2_ref, ba2_ref,
                DsT_ref, DpT_ref,
                next_ref, alpha_ref,
                h2_ref, lcat_ref, lp_ref, btsT_ref, btpT_ref, sprev_ref):
    i = pl.program_id(0)
    p = jax.lax.rem(i, 2)
    pp = 1 - p
    R = state_ref.shape[0]

    def search2(carry):
        lo_s, hi_s, lo_p, hi_p = carry
        for _ in range(4):
            lo_s, hi_s = _search_step(btsT_ref[pp], k_s, lo_s, hi_s)
            lo_p, hi_p = _search_step(btpT_ref[pp], k_p, lo_p, hi_p)
        return lo_s, hi_s, lo_p, hi_p

    s = state_ref[...]
    h1 = s @ W1s_ref[...] + action_ref[...] @ W1a_ref[...] + b1_ref[...]
    h1 = jnp.maximum(h1, 0.0)

    carry = _search_init(R) + _search_init(R)

    def bodyA(j, carry):
        c = j * _CH
        h2_ref[:, pl.ds(c, _CH)] = jnp.maximum(
            h1 @ W2_ref[:, pl.ds(c, _CH)] + b2_ref[:, pl.ds(c, _CH)], 0.0)
        return search2(carry)

    carry = jax.lax.fori_loop(0, 4, bodyA, carry)
    h2 = h2_ref[...]

    def bodyB(j, carry):
        c = j * _CH
        lcat_ref[p, :, pl.ds(c, _CH)] = (
            h2 @ Wcat_ref[:, pl.ds(c, _CH)] + bcat_ref[:, pl.ds(c, _CH)])
        return search2(carry)

    carry = jax.lax.fori_loop(0, 3, bodyB, carry)
    ah = jnp.maximum(lcat_ref[p, :, _KS:], 0.0)

    def bodyC(j, carry):
        c = j * _CH
        lp_ref[p, :, pl.ds(c, _CH)] = (
            ah @ Wa2_ref[:, pl.ds(c, _CH)] + ba2_ref[:, pl.ds(c, _CH)])
        return search2(carry)

    lo_s, _, lo_p, _ = jax.lax.fori_loop(0, 1, bodyC, carry)

    ls_prev = lcat_ref[pp, :, :_KS]
    lp_prev = lp_ref[pp]
    bits_s = jax.lax.bitcast_convert_type(jnp.abs(ls_prev), jnp.int32)
    bits_p = jax.lax.bitcast_convert_type(jnp.abs(lp_prev), jnp.int32)
    alpha_s = jnp.where(bits_s >= lo_s.T, ls_prev, 0.0)
    alpha_p = jnp.where(bits_p >= lo_p.T, lp_prev, 0.0)

    delta = alpha_s @ DsT_ref[...] + alpha_p @ DpT_ref[...]
    next_ref[...] = sprev_ref[pp] + delta
    alpha_ref[:, :_KS] = alpha_s
    alpha_ref[:, _KS:] = alpha_p

    sprev_ref[p] = s
    btsT_ref[p] = jax.lax.bitcast_convert_type(
        jnp.abs(lcat_ref[p, :, :_KS]), jnp.int32).T
    btpT_ref[p] = jax.lax.bitcast_convert_type(
        jnp.abs(lp_ref[p]), jnp.int32).T


def kernel(state, action, W1, b1, W2, b2, Ws, bs, Wa1, ba1, Wa2, ba2, Ds, Dp):
    B, S = state.shape
    A = action.shape[1]
    H = W1.shape[1]
    KS = Ws.shape[1]
    KP = Wa2.shape[1]
    AD = Wa1.shape[1]
    R = 256
    assert B % R == 0
    nblk = B // R

    W1s = W1[:S]
    W1a = W1[S:]
    Wcat = jnp.concatenate([Ws, Wa1], axis=1)
    bcat = jnp.concatenate([bs, ba1]).reshape(1, KS + AD)
    DsT = Ds.T
    DpT = Dp.T

    full = lambda shape: pl.BlockSpec(shape, lambda i: (0, 0))
    last = nblk - 1
    cur = lambda i: (jnp.minimum(i, last), 0)
    prev = lambda i: (jnp.maximum(i - 1, 0), 0)
    grid = (nblk + 1,)
    in_specs = [
            pl.BlockSpec((R, S), cur),
            pl.BlockSpec((R, A), cur),
            full((S, H)),
            full((A, H)),
            full((1, H)),
            full((H, H)),
            full((1, H)),
            full((H, KS + AD)),
            full((1, KS + AD)),
            full((AD, KP)),
            full((1, KP)),
            full((KS, S)),
            full((KP, S)),
    ]
    out_specs = [
        pl.BlockSpec((R, S), prev),
        pl.BlockSpec((R, KS + KP), prev),
    ]

    next_state, alpha = pl.pallas_call(
        functools.partial(_fused_body, 64, 64, nblk),
        grid=grid,
        in_specs=in_specs,
        out_specs=out_specs,
        out_shape=[
            jax.ShapeDtypeStruct((B, S), jnp.float32),
            jax.ShapeDtypeStruct((B, KS + KP), jnp.float32),
        ],
        scratch_shapes=[
            pltpu.VMEM((R, H), jnp.float32),
            pltpu.VMEM((2, R, KS + AD), jnp.float32),
            pltpu.VMEM((2, R, KP), jnp.float32),
            pltpu.VMEM((2, KS, R), jnp.int32),
            pltpu.VMEM((2, KP, R), jnp.int32),
            pltpu.VMEM((2, R, S), jnp.float32),
        ],
        compiler_params=pltpu.CompilerParams(
            dimension_semantics=("arbitrary",),
        ),
    )(state, action, W1s, W1a, b1.reshape(1, H), W2, b2.reshape(1, H),
      Wcat, bcat, Wa2, ba2.reshape(1, KP), DsT, DpT)
    return (next_state, alpha)


# speedup vs baseline: 1.1244x; 1.0552x over previous
import functools

import jax
import jax.numpy as jnp
from jax.experimental import pallas as pl
from jax.experimental.pallas import tpu as pltpu

_KS = 1024
_KP = 512


def _topk_thresholds_t(bits_aT, k_a, bits_bT, k_b):
    cols = bits_aT.shape[1]

    def init():
        return (jnp.zeros((1, cols), jnp.int32),
                jnp.full((1, cols), 0x7F7FFFFF, jnp.int32))

    def step(bits, k, lo, hi):
        mid = lo + (hi - lo + 1) // 2
        cnt = jnp.sum((bits >= mid).astype(jnp.int32), axis=0, keepdims=True)
        ok = cnt >= k
        return jnp.where(ok, mid, lo), jnp.where(ok, hi, mid - 1)

    def body(_, carry):
        lo_a, hi_a, lo_b, hi_b = carry
        lo_a, hi_a = step(bits_aT, k_a, lo_a, hi_a)
        lo_b, hi_b = step(bits_bT, k_b, lo_b, hi_b)
        return lo_a, hi_a, lo_b, hi_b

    lo_a, _, lo_b, _ = jax.lax.fori_loop(0, 31, body, init() + init(),
                                         unroll=8)
    return lo_a, lo_b


def _fused_body(k_s, k_p,
                state_ref, action_ref, W1s_ref, W1a_ref, b1_ref, W2_ref,
                b2_ref, Ws_ref, bs_ref, Wa1_ref, ba1_ref, Wa2_ref, ba2_ref,
                DsT_ref, DpT_ref, next_ref, alpha_ref):
    s = state_ref[...]
    h = s @ W1s_ref[...] + action_ref[...] @ W1a_ref[...] + b1_ref[...]
    h = jnp.maximum(h, 0.0)
    h = jnp.maximum(h @ W2_ref[...] + b2_ref[...], 0.0)

    ls = h @ Ws_ref[...] + bs_ref[...]
    ah = jnp.maximum(h @ Wa1_ref[...] + ba1_ref[...], 0.0)
    lp = ah @ Wa2_ref[...] + ba2_ref[...]

    bits_s = jax.lax.bitcast_convert_type(jnp.abs(ls), jnp.int32)
    bits_p = jax.lax.bitcast_convert_type(jnp.abs(lp), jnp.int32)
    lo_s, lo_p = _topk_thresholds_t(bits_s.T, k_s, bits_p.T, k_p)

    alpha_s = jnp.where(bits_s >= lo_s.T, ls, 0.0)
    alpha_p = jnp.where(bits_p >= lo_p.T, lp, 0.0)

    delta = alpha_s @ DsT_ref[...] + alpha_p @ DpT_ref[...]
    next_ref[...] = s + delta
    alpha_ref[:, :_KS] = alpha_s
    alpha_ref[:, _KS:] = alpha_p


def kernel(state, action, W1, b1, W2, b2, Ws, bs, Wa1, ba1, Wa2, ba2, Ds, Dp):
    B, S = state.shape
    A = action.shape[1]
    H = W1.shape[1]
    KS = Ws.shape[1]
    KP = Wa2.shape[1]
    AD = Wa1.shape[1]
    R = 512
    assert B % R == 0

    W1s = W1[:S]
    W1a = W1[S:]
    DsT = Ds.T
    DpT = Dp.T

    full = lambda shape: pl.BlockSpec(shape, lambda i: (0, 0))
    grid_spec = pl.GridSpec(
        grid=(B // R,),
        in_specs=[
            pl.BlockSpec((R, S), lambda i: (i, 0)),
            pl.BlockSpec((R, A), lambda i: (i, 0)),
            full((S, H)),
            full((A, H)),
            full((1, H)),
            full((H, H)),
            full((1, H)),
            full((H, KS)),
            full((1, KS)),
            full((H, AD)),
            full((1, AD)),
            full((AD, KP)),
            full((1, KP)),
            full((KS, S)),
            full((KP, S)),
        ],
        out_specs=[
            pl.BlockSpec((R, S), lambda i: (i, 0)),
            pl.BlockSpec((R, KS + KP), lambda i: (i, 0)),
        ],
    )

    next_state, alpha = pl.pallas_call(
        functools.partial(_fused_body, 64, 64),
        grid_spec=grid_spec,
        out_shape=[
            jax.ShapeDtypeStruct((B, S), jnp.float32),
            jax.ShapeDtypeStruct((B, KS + KP), jnp.float32),
        ],
        compiler_params=pltpu.CompilerParams(
            dimension_semantics=("arbitrary",),
        ),
    )(state, action, W1s, W1a, b1.reshape(1, H), W2, b2.reshape(1, H),
      Ws, bs.reshape(1, KS), Wa1, ba1.reshape(1, AD), Wa2, ba2.reshape(1, KP),
      DsT, DpT)
    return (next_state, alpha)


# speedup vs baseline: 1.1465x; 1.0196x over previous
import functools

import jax
import jax.numpy as jnp
from jax.experimental import pallas as pl
from jax.experimental.pallas import tpu as pltpu

_KS = 1024
_KP = 512


def _topk_thresholds_t(bits_aT, k_a, bits_bT, k_b):
    cols = bits_aT.shape[1]

    def init():
        return (jnp.zeros((1, cols), jnp.int32),
                jnp.full((1, cols), 0x7F7FFFFF, jnp.int32))

    def step(bits, k, lo, hi):
        mid = lo + (hi - lo + 1) // 2
        cnt = jnp.sum((bits >= mid).astype(jnp.int32), axis=0, keepdims=True)
        ok = cnt >= k
        return jnp.where(ok, mid, lo), jnp.where(ok, hi, mid - 1)

    def body(_, carry):
        lo_a, hi_a, lo_b, hi_b = carry
        lo_a, hi_a = step(bits_aT, k_a, lo_a, hi_a)
        lo_b, hi_b = step(bits_bT, k_b, lo_b, hi_b)
        return lo_a, hi_a, lo_b, hi_b

    lo_a, _, lo_b, _ = jax.lax.fori_loop(0, 31, body, init() + init(),
                                         unroll=4)
    return lo_a, lo_b


def _fused_body(k_s, k_p,
                state_ref, action_ref, W1s_ref, W1a_ref, b1_ref, W2_ref,
                b2_ref, Ws_ref, bs_ref, Wa1_ref, ba1_ref, Wa2_ref, ba2_ref,
                DsT_ref, DpT_ref, next_ref, alpha_ref):
    s = state_ref[...]
    h = s @ W1s_ref[...] + action_ref[...] @ W1a_ref[...] + b1_ref[...]
    h = jnp.maximum(h, 0.0)
    h = jnp.maximum(h @ W2_ref[...] + b2_ref[...], 0.0)

    ls = h @ Ws_ref[...] + bs_ref[...]
    ah = jnp.maximum(h @ Wa1_ref[...] + ba1_ref[...], 0.0)
    lp = ah @ Wa2_ref[...] + ba2_ref[...]

    bits_s = jax.lax.bitcast_convert_type(jnp.abs(ls), jnp.int32)
    bits_p = jax.lax.bitcast_convert_type(jnp.abs(lp), jnp.int32)
    lo_s, lo_p = _topk_thresholds_t(bits_s.T, k_s, bits_p.T, k_p)

    alpha_s = jnp.where(bits_s >= lo_s.T, ls, 0.0)
    alpha_p = jnp.where(bits_p >= lo_p.T, lp, 0.0)

    delta = alpha_s @ DsT_ref[...] + alpha_p @ DpT_ref[...]
    next_ref[...] = s + delta
    alpha_ref[:, :_KS] = alpha_s
    alpha_ref[:, _KS:] = alpha_p


def kernel(state, action, W1, b1, W2, b2, Ws, bs, Wa1, ba1, Wa2, ba2, Ds, Dp):
    B, S = state.shape
    A = action.shape[1]
    H = W1.shape[1]
    KS = Ws.shape[1]
    KP = Wa2.shape[1]
    AD = Wa1.shape[1]
    R = 512
    assert B % R == 0

    W1s = W1[:S]
    W1a = W1[S:]
    DsT = Ds.T
    DpT = Dp.T

    full = lambda shape: pl.BlockSpec(shape, lambda i: (0, 0))
    grid_spec = pl.GridSpec(
        grid=(B // R,),
        in_specs=[
            pl.BlockSpec((R, S), lambda i: (i, 0)),
            pl.BlockSpec((R, A), lambda i: (i, 0)),
            full((S, H)),
            full((A, H)),
            full((1, H)),
            full((H, H)),
            full((1, H)),
            full((H, KS)),
            full((1, KS)),
            full((H, AD)),
            full((1, AD)),
            full((AD, KP)),
            full((1, KP)),
            full((KS, S)),
            full((KP, S)),
        ],
        out_specs=[
            pl.BlockSpec((R, S), lambda i: (i, 0)),
            pl.BlockSpec((R, KS + KP), lambda i: (i, 0)),
        ],
    )

    next_state, alpha = pl.pallas_call(
        functools.partial(_fused_body, 64, 64),
        grid_spec=grid_spec,
        out_shape=[
            jax.ShapeDtypeStruct((B, S), jnp.float32),
            jax.ShapeDtypeStruct((B, KS + KP), jnp.float32),
        ],
        compiler_params=pltpu.CompilerParams(
            dimension_semantics=("arbitrary",),
        ),
    )(state, action, W1s, W1a, b1.reshape(1, H), W2, b2.reshape(1, H),
      Ws, bs.reshape(1, KS), Wa1, ba1.reshape(1, AD), Wa2, ba2.reshape(1, KP),
      DsT, DpT)
    return (next_state, alpha)
